# baseline (speedup 1.0000x reference)
import functools

import jax
import jax.numpy as jnp
from jax import lax
from jax.experimental import pallas as pl
from jax.experimental.pallas import tpu as pltpu
from jax.experimental.pallas import tpu_sc as plsc

_VOCAB = 1000000
_EMB_DIM = 64
_BATCH = 4096
_HIST = 50
_B = _BATCH * _HIST

_info = plsc.get_sparse_core_info()
_NC = _info.num_cores
_NS = _info.num_subcores
_NW = _NC * _NS
_BC = _BATCH // 128
_NBLK = _HIST * _BC
_BLK_PER_W = _NBLK // _NW
_NBUF = 2

_mesh = plsc.VectorSubcoreMesh(core_axis_name="c", subcore_axis_name="s")


@functools.partial(
    pl.kernel,
    mesh=_mesh,
    out_type=jax.ShapeDtypeStruct((_HIST, 8, _BC, 8, 128), jnp.float32),
    scratch_types=[
        [pltpu.VMEM((128,), jnp.int32) for _ in range(_NBUF)],
        [pltpu.VMEM((128, _EMB_DIM), jnp.float32) for _ in range(_NBUF)],
        [pltpu.VMEM((8, 8, 128), jnp.float32) for _ in range(_NBUF)],
        [pltpu.SemaphoreType.DMA for _ in range(_NBUF)],
        [pltpu.SemaphoreType.DMA for _ in range(_NBUF)],
        [pltpu.SemaphoreType.DMA for _ in range(_NBUF)],
    ],
    compiler_params=pltpu.CompilerParams(
        use_tc_tiling_on_sc=False, needs_layout_passes=False
    ),
)
def _lookup_kernel(table_hbm, idxt_hbm, out_hbm, ib, gb, lb, isem, gsem, wsem):
    wid = lax.axis_index("s") * _NC + lax.axis_index("c")
    blk0 = wid * _BLK_PER_W

    def start_fetch(k):
        d = k % _NBUF
        blk = blk0 + k
        h = blk // _BC
        bc = blk % _BC
        pltpu.async_copy(idxt_hbm.at[pl.ds(h * _BATCH + bc * 128, 128)], ib[d], isem[d])

    def start_gather(k):
        d = k % _NBUF
        pltpu.make_async_copy(idxt_hbm.at[pl.ds(0, 128)], ib[d], isem[d]).wait()
        pltpu.async_copy(table_hbm.at[ib[d]], gb[d], gsem[d])

    start_fetch(0)
    start_gather(0)
    start_fetch(1)

    lane = lax.broadcasted_iota(jnp.int32, (16,), 0)

    for k in range(_BLK_PER_W):
        d = k % _NBUF
        blk = blk0 + k
        h = blk // _BC
        bc = blk % _BC
        if k + 1 < _BLK_PER_W:
            start_gather(k + 1)
        pltpu.make_async_copy(table_hbm.at[ib[d]], gb[d], gsem[d]).wait()
        if k + 2 < _BLK_PER_W:
            start_fetch(k + 2)
        if k >= _NBUF:
            pb = blk0 + k - _NBUF
            pltpu.make_async_copy(
                lb[d], out_hbm.at[pb // _BC, :, pb % _BC], wsem[d]
            ).wait()

        def trans_body(e, carry):
            a = e // 8
            r = e % 8
            for cg in range(8):
                vals = plsc.load_gather(
                    gb[d], [lane + cg * 16, jnp.full((16,), 0, jnp.int32) + e]
                )
                lb[d][a, r, pl.ds(cg * 16, 16)] = vals
            return carry

        lax.fori_loop(0, _EMB_DIM, trans_body, 0)
        pltpu.async_copy(lb[d], out_hbm.at[h, :, bc], wsem[d])

    for k in range(_BLK_PER_W - _NBUF, _BLK_PER_W):
        d = k % _NBUF
        pb = blk0 + k
        pltpu.make_async_copy(
            lb[d], out_hbm.at[pb // _BC, :, pb % _BC], wsem[d]
        ).wait()


def kernel(x, table):
    idxt = x.T.reshape(-1)
    lin = _lookup_kernel(table, idxt)
    out = lin.transpose((2, 4, 0, 1, 3)).reshape(_BATCH, _HIST, _EMB_DIM)
    return out


# speedup vs baseline: 1.5358x; 1.5358x over previous
import functools

import jax
import jax.numpy as jnp
from jax import lax
from jax.experimental import pallas as pl
from jax.experimental.pallas import tpu as pltpu
from jax.experimental.pallas import tpu_sc as plsc

_VOCAB = 1000000
_EMB_DIM = 64
_BATCH = 4096
_HIST = 50
_B = _BATCH * _HIST

_info = plsc.get_sparse_core_info()
_NC = _info.num_cores
_NS = _info.num_subcores
_NW = _NC * _NS
_B_PER_W = _B // _NW
_CHUNK = 400
_NBUF = 2
_N_CHUNKS = _B_PER_W // _CHUNK
_HCHUNK = _CHUNK // 2

_mesh = plsc.VectorSubcoreMesh(core_axis_name="c", subcore_axis_name="s")


@functools.partial(
    pl.kernel,
    mesh=_mesh,
    out_type=jax.ShapeDtypeStruct((_B, _EMB_DIM), jnp.float32),
    scratch_types=[
        pltpu.VMEM((8, 800), jnp.int32),
        [pltpu.VMEM((_CHUNK, _EMB_DIM), jnp.float32) for _ in range(_NBUF)],
        [pltpu.SemaphoreType.DMA for _ in range(_NBUF)],
        [pltpu.SemaphoreType.DMA for _ in range(_NBUF)],
    ],
    compiler_params=pltpu.CompilerParams(
        use_tc_tiling_on_sc=True, needs_layout_passes=False
    ),
)
def _gather_kernel(table_hbm, idx_hbm, out_hbm, ib, rb, gsem, wsem):
    wid = lax.axis_index("s") * _NC + lax.axis_index("c")
    base = pl.multiple_of(wid * _B_PER_W, 1024)
    pltpu.sync_copy(idx_hbm.at[wid], ib)

    def fire_chunk(c):
        d = c % _NBUF

        def group(g, carry):
            j = c * _CHUNK + g * 16
            vvec = ib[j // 800, pl.ds(j % 800, 16)]
            for lane in range(16):
                v = vvec[lane]
                pltpu.async_copy(
                    table_hbm.at[pl.ds(v, 1), :],
                    rb[d].at[pl.ds(g * 16 + lane, 1), :],
                    gsem[d],
                )
            return carry

        lax.fori_loop(0, _CHUNK // 16, group, 0)

    def drain_chunk(c):
        d = c % _NBUF

        def group(g, carry):
            pltpu.make_async_copy(
                table_hbm.at[pl.ds(0, 1), :], rb[d].at[pl.ds(0, 1), :], gsem[d]
            ).wait()
            return carry

        lax.fori_loop(0, _CHUNK, group, 0)

    for c in range(_N_CHUNKS):
        d = c % _NBUF
        fire_chunk(c)
        drain_chunk(c)
        pltpu.sync_copy(rb[d], out_hbm.at[pl.ds(base + c * _CHUNK, _CHUNK)])


def kernel(x, table):
    out2 = _gather_kernel(table, x.reshape(_NW, 8, _B_PER_W // 8))
    return out2.reshape(_BATCH, _HIST, _EMB_DIM)


# speedup vs baseline: 1.7889x; 1.1648x over previous
import functools

import jax
import jax.numpy as jnp
from jax import lax
from jax.experimental import pallas as pl
from jax.experimental.pallas import tpu as pltpu
from jax.experimental.pallas import tpu_sc as plsc

_VOCAB = 1000000
_EMB_DIM = 64
_BATCH = 4096
_HIST = 50
_B = _BATCH * _HIST

_info = plsc.get_sparse_core_info()
_NC = _info.num_cores
_NS = _info.num_subcores
_NW = _NC * _NS
_B_PER_W = _B // _NW
_CHUNK = 400
_NBUF = 2
_N_CHUNKS = _B_PER_W // _CHUNK
_HCHUNK = _CHUNK // 2

_mesh = plsc.VectorSubcoreMesh(core_axis_name="c", subcore_axis_name="s")


@functools.partial(
    pl.kernel,
    mesh=_mesh,
    out_type=jax.ShapeDtypeStruct((_BATCH, _HIST, _EMB_DIM), jnp.float32),
    scratch_types=[
        pltpu.VMEM((8, 800), jnp.int32),
        [pltpu.VMEM((8, _HIST, _EMB_DIM), jnp.float32) for _ in range(_NBUF)],
        [pltpu.SemaphoreType.DMA for _ in range(_NBUF)],
        [pltpu.SemaphoreType.DMA for _ in range(_NBUF)],
    ],
    compiler_params=pltpu.CompilerParams(
        use_tc_tiling_on_sc=True, needs_layout_passes=False
    ),
)
def _gather_kernel(table_hbm, idx_hbm, out_hbm, ib, rb, gsem, wsem):
    wid = lax.axis_index("s") * _NC + lax.axis_index("c")
    base = pl.multiple_of(wid * _B_PER_W, 1024)
    bbase = pl.multiple_of(wid * (_BATCH // _NW), 8)
    pltpu.sync_copy(idx_hbm.at[wid], ib)

    def fire_chunk(c):
        d = c % _NBUF

        def group(g, carry):
            j = c * _CHUNK + g * 16
            vvec = ib[j // 800, pl.ds(j % 800, 16)]
            for lane in range(16):
                v = vvec[lane]
                row = g * 16 + lane
                pltpu.async_copy(
                    table_hbm.at[pl.ds(v, 1), :],
                    rb[d].at[row // _HIST, pl.ds(row % _HIST, 1), :],
                    gsem[d],
                )
            return carry

        lax.fori_loop(0, _CHUNK // 16, group, 0)

    def drain_chunk(c):
        d = c % _NBUF

        def group(g, carry):
            pltpu.make_async_copy(
                out_hbm.at[pl.ds(0, 8)], rb[d], gsem[d]
            ).wait()
            return carry

        lax.fori_loop(0, 1, group, 0)

    fire_chunk(0)
    for c in range(_N_CHUNKS):
        d = c % _NBUF
        if c + 1 < _N_CHUNKS:
            if c >= 1:
                pltpu.make_async_copy(
                    rb[(c + 1) % _NBUF],
                    out_hbm.at[pl.ds(bbase + (c - 1) * 8, 8)],
                    wsem[(c + 1) % _NBUF],
                ).wait()
            fire_chunk(c + 1)
        drain_chunk(c)
        pltpu.async_copy(rb[d], out_hbm.at[pl.ds(bbase + c * 8, 8)], wsem[d])

    for c in range(_N_CHUNKS - _NBUF, _N_CHUNKS):
        d = c % _NBUF
        pltpu.make_async_copy(
            rb[d], out_hbm.at[pl.ds(bbase + c * 8, 8)], wsem[d]
        ).wait()


def kernel(x, table):
    return _gather_kernel(table, x.reshape(_NW, 8, _B_PER_W // 8))
